# baseline (device time: 35805 ns/iter reference)
import jax
import jax.numpy as jnp
from jax import lax
from jax.experimental import pallas as pl
from jax.experimental.pallas import tpu as pltpu

N_DEV = 4
E_LOCAL = 4


def kernel(x, router_W, route_idx, expert_W, shared_W):
    n, d = x.shape
    h = shared_W.shape[1]

    def body(x_ref, rw_ref, idx_ref, ew_ref, sw_ref, out_ref,
             comm_ref, send_sems, recv_sems):
        my = lax.axis_index("i")

        bar = pltpu.get_barrier_semaphore()
        for k in range(1, N_DEV):
            pl.semaphore_signal(
                bar, inc=1,
                device_id=((my + k) % N_DEV,),
                device_id_type=pl.DeviceIdType.MESH,
            )
        pl.semaphore_wait(bar, N_DEV - 1)

        xv = x_ref[...]
        idx = idx_ref[...]

        scores = jnp.dot(xv, rw_ref[...], preferred_element_type=jnp.float32)
        m = jnp.max(scores, axis=-1, keepdims=True)
        p = jnp.exp(scores - m)
        p = p / jnp.sum(p, axis=-1, keepdims=True)
        iota = lax.broadcasted_iota(jnp.int32, scores.shape, 1)
        top1 = jnp.sum(jnp.where(iota == idx, p, 0.0), axis=-1,
                       keepdims=True)

        acc = None
        for le in range(E_LOCAL):
            eg = my * E_LOCAL + le
            w = jnp.where(idx == eg, top1, 0.0)
            contrib = jnp.dot(xv * w, ew_ref[le],
                              preferred_element_type=jnp.float32)
            acc = contrib if acc is None else acc + contrib
        comm_ref[pl.ds(my, 1)] = acc[None]

        sends = []
        for k in range(1, N_DEV):
            t = (my + k) % N_DEV
            rd = pltpu.make_async_remote_copy(
                src_ref=comm_ref.at[my],
                dst_ref=comm_ref.at[my],
                send_sem=send_sems.at[k - 1],
                recv_sem=recv_sems.at[my],
                device_id=(t,),
                device_id_type=pl.DeviceIdType.MESH,
            )
            rd.start()
            sends.append(rd)

        shared = jnp.dot(xv, sw_ref[...], preferred_element_type=jnp.float32)

        for k in range(1, N_DEV):
            s = (my + k) % N_DEV
            rr = pltpu.make_async_remote_copy(
                src_ref=comm_ref.at[s],
                dst_ref=comm_ref.at[s],
                send_sem=send_sems.at[0],
                recv_sem=recv_sems.at[s],
                device_id=(s,),
                device_id_type=pl.DeviceIdType.MESH,
            )
            rr.wait_recv()

        out_ref[...] = (shared + comm_ref[0] + comm_ref[1]
                        + comm_ref[2] + comm_ref[3])

        for rd in sends:
            rd.wait_send()

    return pl.pallas_call(
        body,
        out_shape=jax.ShapeDtypeStruct((n, h), jnp.float32),
        in_specs=[pl.BlockSpec(memory_space=pltpu.VMEM)] * 5,
        out_specs=pl.BlockSpec(memory_space=pltpu.VMEM),
        scratch_shapes=[
            pltpu.VMEM((N_DEV, n, h), jnp.float32),
            pltpu.SemaphoreType.DMA((N_DEV - 1,)),
            pltpu.SemaphoreType.DMA((N_DEV,)),
        ],
        compiler_params=pltpu.CompilerParams(collective_id=0),
    )(x, router_W, route_idx, expert_W, shared_W)


# device time: 26660 ns/iter; 1.3430x vs baseline; 1.3430x over previous
import jax
import jax.numpy as jnp
from jax import lax
from jax.experimental import pallas as pl
from jax.experimental.pallas import tpu as pltpu

N_DEV = 4
E_LOCAL = 4


def kernel(x, router_W, route_idx, expert_W, shared_W):
    n, d = x.shape
    h = shared_W.shape[1]
    q = n // N_DEV

    def body(x_ref, rw_ref, idx_ref, ew_ref, sw_ref, out_ref,
             stage_ref, rs_ref, ag_ref, rs_send, rs_recv, ag_send, ag_recv):
        my = lax.axis_index("i")

        bar = pltpu.get_barrier_semaphore()
        for k in range(1, N_DEV):
            pl.semaphore_signal(
                bar, inc=1,
                device_id=((my + k) % N_DEV,),
                device_id_type=pl.DeviceIdType.MESH,
            )
        pl.semaphore_wait(bar, N_DEV - 1)

        xv = x_ref[...]
        idx = idx_ref[...]

        scores = jnp.dot(xv, rw_ref[...], preferred_element_type=jnp.float32)
        m = jnp.max(scores, axis=-1, keepdims=True)
        p = jnp.exp(scores - m)
        p = p / jnp.sum(p, axis=-1, keepdims=True)
        iota = lax.broadcasted_iota(jnp.int32, scores.shape, 1)
        top1 = jnp.sum(jnp.where(iota == idx, p, 0.0), axis=-1,
                       keepdims=True)

        acc = None
        for le in range(E_LOCAL):
            eg = my * E_LOCAL + le
            w = jnp.where(idx == eg, top1, 0.0)
            contrib = jnp.dot(xv * w, ew_ref[le],
                              preferred_element_type=jnp.float32)
            acc = contrib if acc is None else acc + contrib
        stage_ref[...] = acc.reshape(N_DEV, q, h)

        rs_list = []
        for k in range(1, N_DEV):
            t = (my + k) % N_DEV
            rd = pltpu.make_async_remote_copy(
                src_ref=stage_ref.at[t],
                dst_ref=rs_ref.at[my],
                send_sem=rs_send.at[k - 1],
                recv_sem=rs_recv.at[my],
                device_id=(t,),
                device_id_type=pl.DeviceIdType.MESH,
            )
            rd.start()
            rs_list.append(rd)

        shared = jnp.dot(xv, sw_ref[...], preferred_element_type=jnp.float32)

        rs_ref[pl.ds(my, 1)] = stage_ref[pl.ds(my, 1)]

        for k in range(1, N_DEV):
            s = (my + k) % N_DEV
            pltpu.make_async_remote_copy(
                src_ref=rs_ref.at[s],
                dst_ref=rs_ref.at[s],
                send_sem=rs_send.at[0],
                recv_sem=rs_recv.at[s],
                device_id=(s,),
                device_id_type=pl.DeviceIdType.MESH,
            ).wait_recv()

        red = rs_ref[0] + rs_ref[1] + rs_ref[2] + rs_ref[3]
        ag_ref[pl.ds(my, 1)] = red[None]

        ag_list = []
        for k in range(1, N_DEV):
            t = (my + k) % N_DEV
            rd = pltpu.make_async_remote_copy(
                src_ref=ag_ref.at[my],
                dst_ref=ag_ref.at[my],
                send_sem=ag_send.at[k - 1],
                recv_sem=ag_recv.at[my],
                device_id=(t,),
                device_id_type=pl.DeviceIdType.MESH,
            )
            rd.start()
            ag_list.append(rd)

        for k in range(1, N_DEV):
            s = (my + k) % N_DEV
            pltpu.make_async_remote_copy(
                src_ref=ag_ref.at[s],
                dst_ref=ag_ref.at[s],
                send_sem=ag_send.at[0],
                recv_sem=ag_recv.at[s],
                device_id=(s,),
                device_id_type=pl.DeviceIdType.MESH,
            ).wait_recv()

        out_ref[...] = ag_ref[...].reshape(n, h) + shared

        for rd in rs_list + ag_list:
            rd.wait_send()

    return pl.pallas_call(
        body,
        out_shape=jax.ShapeDtypeStruct((n, h), jnp.float32),
        in_specs=[pl.BlockSpec(memory_space=pltpu.VMEM)] * 5,
        out_specs=pl.BlockSpec(memory_space=pltpu.VMEM),
        scratch_shapes=[
            pltpu.VMEM((N_DEV, q, h), jnp.float32),
            pltpu.VMEM((N_DEV, q, h), jnp.float32),
            pltpu.VMEM((N_DEV, q, h), jnp.float32),
            pltpu.SemaphoreType.DMA((N_DEV - 1,)),
            pltpu.SemaphoreType.DMA((N_DEV,)),
            pltpu.SemaphoreType.DMA((N_DEV - 1,)),
            pltpu.SemaphoreType.DMA((N_DEV,)),
        ],
        compiler_params=pltpu.CompilerParams(collective_id=0),
    )(x, router_W, route_idx, expert_W, shared_W)


# device time: 19221 ns/iter; 1.8628x vs baseline; 1.3870x over previous
import jax
import jax.numpy as jnp
from jax import lax
from jax.experimental import pallas as pl
from jax.experimental.pallas import tpu as pltpu

N_DEV = 4
E_LOCAL = 4
N_WAVE = 2
CAP = 64


def kernel(x, router_W, route_idx, expert_W, shared_W):
    n, d = x.shape
    h = shared_W.shape[1]
    q = n // N_DEV
    hh = h // N_WAVE

    def body(x_ref, rw_ref, idx_ref, ew_ref, sw_ref, out_ref,
             stage_ref, rs_ref, ag_ref, rs_send, rs_recv, ag_send, ag_recv):
        my = lax.axis_index("i")

        bar = pltpu.get_barrier_semaphore()
        for k in range(1, N_DEV):
            pl.semaphore_signal(
                bar, inc=1,
                device_id=((my + k) % N_DEV,),
                device_id_type=pl.DeviceIdType.MESH,
            )

        xv = x_ref[...]
        idx = idx_ref[...]

        scores = jnp.dot(xv, rw_ref[...], preferred_element_type=jnp.float32)
        m = jnp.max(scores, axis=-1, keepdims=True)
        p = jnp.exp(scores - m)
        p = p / jnp.sum(p, axis=-1, keepdims=True)
        iota = lax.broadcasted_iota(jnp.int32, scores.shape, 1)
        top1 = jnp.sum(jnp.where(iota == idx, p, 0.0), axis=-1,
                       keepdims=True)

        xp = xv * top1
        xw = jnp.concatenate(
            [jnp.where(idx == my * E_LOCAL + le, xp, 0.0)
             for le in range(E_LOCAL)],
            axis=1,
        ).astype(jnp.bfloat16)
        ewb = ew_ref[...].astype(jnp.bfloat16).reshape(E_LOCAL * d, h)

        io_q0 = lax.broadcasted_iota(jnp.int32, (q, q), 0)
        io_q1 = lax.broadcasted_iota(jnp.int32, (q, q), 1)
        tri = (io_q0 < io_q1).astype(jnp.float32)
        diag = (io_q0 == io_q1).astype(jnp.float32)
        io_cq = lax.broadcasted_iota(jnp.int32, (CAP, q), 0)
        io_qc = lax.broadcasted_iota(jnp.int32, (q, CAP), 1)

        def row_forms(sel):
            self32 = sel.astype(jnp.float32)
            rank_row = jnp.sum(self32 * tri, axis=0, keepdims=True)
            sel_row = jnp.sum(self32 * diag, axis=0, keepdims=True)
            return sel_row, rank_row.astype(jnp.int32)

        pl.semaphore_wait(bar, N_DEV - 1)

        def rs_desc(t, w):
            return pltpu.make_async_remote_copy(
                src_ref=stage_ref.at[t, :, pl.ds(w * hh, hh)],
                dst_ref=rs_ref.at[my, :, pl.ds(w * hh, hh)],
                send_sem=rs_send.at[t, w],
                recv_sem=rs_recv.at[my, w],
                device_id=(t,),
                device_id_type=pl.DeviceIdType.MESH,
            )

        def rs_wait_desc(s, w):
            return pltpu.make_async_remote_copy(
                src_ref=rs_ref.at[s, :, pl.ds(w * hh, hh)],
                dst_ref=rs_ref.at[s, :, pl.ds(w * hh, hh)],
                send_sem=rs_send.at[0, w],
                recv_sem=rs_recv.at[s, w],
                device_id=(s,),
                device_id_type=pl.DeviceIdType.MESH,
            )

        def ag_desc(w, k):
            return pltpu.make_async_remote_copy(
                src_ref=ag_ref.at[my, :, pl.ds(w * hh, hh)],
                dst_ref=ag_ref.at[my, :, pl.ds(w * hh, hh)],
                send_sem=ag_send.at[k, w],
                recv_sem=ag_recv.at[my, w],
                device_id=((my + k + 1) % N_DEV,),
                device_id_type=pl.DeviceIdType.MESH,
            )

        for t in range(N_DEV):
            idx_t = idx[t * q:(t + 1) * q]
            sel = (idx_t // E_LOCAL) == my
            sel_row, rank_row = row_forms(sel)
            gath = jnp.where((io_cq == rank_row) & (sel_row > 0.5), 1.0, 0.0
                             ).astype(jnp.bfloat16)
            xg = jnp.dot(gath, xw[t * q:(t + 1) * q],
                         preferred_element_type=jnp.float32
                         ).astype(jnp.bfloat16)
            compact = jnp.dot(xg, ewb, preferred_element_type=jnp.float32)
            stage_ref[t] = compact.astype(jnp.bfloat16)

            @pl.when(t != my)
            def _():
                rs_desc(t, 0).start()

        for w in range(1, N_WAVE):
            for t in range(N_DEV):
                @pl.when(t != my)
                def _():
                    rs_desc(t, w).start()

        shared = jnp.dot(xv.astype(jnp.bfloat16),
                         sw_ref[...].astype(jnp.bfloat16),
                         preferred_element_type=jnp.float32)
        out_ref[...] = shared
        rs_ref[pl.ds(my, 1)] = stage_ref[pl.ds(my, 1)]

        idx_q = idx_ref[pl.ds(my * q, q), :]
        scat = []
        for s in range(N_DEV):
            sel = (idx_q // E_LOCAL) == s
            _, rank_row = row_forms(sel)
            rank_col = jnp.sum(rank_row.astype(jnp.float32) * diag, axis=1,
                               keepdims=True).astype(jnp.int32)
            scat.append(jnp.where((io_qc == rank_col) & sel, 1.0, 0.0
                                  ).astype(jnp.bfloat16))

        ag_all = []
        reds = []
        for w in range(N_WAVE):
            for k in range(1, N_DEV):
                rs_wait_desc((my + k) % N_DEV, w).wait_recv()
            red = None
            for s in range(N_DEV):
                part = jnp.dot(scat[s],
                               rs_ref[s, :, w * hh:(w + 1) * hh],
                               preferred_element_type=jnp.float32)
                red = part if red is None else red + part
            redb = red.astype(jnp.bfloat16)
            reds.append(red)
            ag_ref[pl.ds(my, 1), :, pl.ds(w * hh, hh)] = redb[None]
            ag_w = [ag_desc(w, k) for k in range(N_DEV - 1)]
            for rd in ag_w:
                rd.start()
            ag_all.append(ag_w)

        for w in range(N_WAVE):
            out_ref[pl.ds(my * q, q), pl.ds(w * hh, hh)] = (
                out_ref[pl.ds(my * q, q), pl.ds(w * hh, hh)] + reds[w])

        for w in range(N_WAVE):
            for k in range(1, N_DEV):
                s = (my + k) % N_DEV
                pltpu.make_async_remote_copy(
                    src_ref=ag_ref.at[s, :, pl.ds(w * hh, hh)],
                    dst_ref=ag_ref.at[s, :, pl.ds(w * hh, hh)],
                    send_sem=ag_send.at[0, w],
                    recv_sem=ag_recv.at[s, w],
                    device_id=(s,),
                    device_id_type=pl.DeviceIdType.MESH,
                ).wait_recv()
                got = ag_ref[pl.ds(s, 1), :, pl.ds(w * hh, hh)][0]
                out_ref[pl.ds(s * q, q), pl.ds(w * hh, hh)] = (
                    out_ref[pl.ds(s * q, q), pl.ds(w * hh, hh)]
                    + got.astype(jnp.float32))

        for t in range(N_DEV):
            for w in range(N_WAVE):
                @pl.when(t != my)
                def _():
                    rs_desc(t, w).wait_send()
        for ag_w in ag_all:
            for rd in ag_w:
                rd.wait_send()

    return pl.pallas_call(
        body,
        out_shape=jax.ShapeDtypeStruct((n, h), jnp.float32),
        in_specs=[pl.BlockSpec(memory_space=pltpu.VMEM)] * 5,
        out_specs=pl.BlockSpec(memory_space=pltpu.VMEM),
        scratch_shapes=[
            pltpu.VMEM((N_DEV, CAP, h), jnp.bfloat16),
            pltpu.VMEM((N_DEV, CAP, h), jnp.bfloat16),
            pltpu.VMEM((N_DEV, q, h), jnp.bfloat16),
            pltpu.SemaphoreType.DMA((N_DEV, N_WAVE)),
            pltpu.SemaphoreType.DMA((N_DEV, N_WAVE)),
            pltpu.SemaphoreType.DMA((N_DEV - 1, N_WAVE)),
            pltpu.SemaphoreType.DMA((N_DEV, N_WAVE)),
        ],
        compiler_params=pltpu.CompilerParams(collective_id=0),
    )(x, router_W, route_idx, expert_W, shared_W)
